# Initial kernel scaffold; baseline (speedup 1.0000x reference)
import jax
import jax.numpy as jnp
from jax.experimental import pallas as pl


def kernel(x, edge_index, W1, b1, W2, b2, W3, b3, Wg, att_src, att_dst, bg, Wc1, bc1, Wc2, bc2):
    raise NotImplementedError("write your pallas kernel here")


# speedup vs baseline: 5.4104x; 5.4104x over previous
import functools

import jax
import jax.numpy as jnp
from jax import lax
from jax.experimental import pallas as pl
from jax.experimental.pallas import tpu as pltpu
from jax.experimental.pallas import tpu_sc as plsc

NN = 50000
HD = 64
NH = 4
NCK = 25000
ACC = 25088
ZR = 1568
TRASH = 25080
CK = 128
BN = 1000
_G16 = CK // 16


def _sc_mesh():
    return plsc.VectorSubcoreMesh(core_axis_name="c", subcore_axis_name="s")


def _loc_idx(dstv, locv, base_n, bound=NCK, trash=TRASH):
    for j in range(_G16):
        v = dstv[pl.ds(j * 16, 16)]
        loc = v - base_n
        ok = (loc >= 0) & (loc < bound)
        locv[pl.ds(j * 16, 16)] = jnp.where(ok, loc, trash)


def _make_deg(ep):
    ep16 = ep // 16
    nch = ep16 // CK

    @functools.partial(
        pl.kernel,
        out_type=jax.ShapeDtypeStruct((4, ACCQ, 16), jnp.float32),
        mesh=_sc_mesh(),
        scratch_types=[
            pltpu.VMEM((CK,), jnp.int32),
            pltpu.VMEM((CK,), jnp.int32),
            pltpu.VMEM((CK, 16), jnp.float32),
            pltpu.VMEM((ZS, 16), jnp.float32),
            pltpu.VMEM_SHARED((ACCQ, 16), jnp.float32),
        ],
    )
    def k(colp, z16, out, dstv, locv, onesv, stage, acc):
        c = lax.axis_index("c")
        s = lax.axis_index("s")
        for e in range(CK):
            onesv[e, pl.ds(0, 16)] = jnp.ones((16,), jnp.float32)
        for r in range(2):
            base_n = c * (2 * NCQ) + r * NCQ
            _q_zero(z16, stage, acc, s)
            plsc.subcore_barrier()

            def body(i, carry):
                e0 = s * ep16 + i * CK
                pltpu.sync_copy(colp.at[pl.ds(e0, CK)], dstv)
                _loc_idx(dstv, locv, base_n, NCQ, TRQ)
                pltpu.sync_copy(onesv, acc.at[locv], add=True)
                return carry

            lax.fori_loop(0, nch, body, 0)
            plsc.subcore_barrier()
            _q_out(acc, stage, out, 2 * c + r, s)
            plsc.subcore_barrier()

    return k


NCQ = 12544
ACCQ = 12672
ZRQ = 792
ZS = 72
ZC = 11
TRQ = 12600


def _q_zero(zq, stage, acc, s):
    pltpu.sync_copy(zq, stage)
    for t in range(ZC):
        pltpu.sync_copy(stage, acc.at[pl.ds(s * ZRQ + t * ZS, ZS)])


def _q_out(acc, stage, out, q, s):
    for t in range(ZC):
        pltpu.sync_copy(acc.at[pl.ds(s * ZRQ + t * ZS, ZS)], stage)
        pltpu.sync_copy(stage, out.at[q, pl.ds(s * ZRQ + t * ZS, ZS)])


def _make_seg(ep):
    ep16 = ep // 16
    nch = ep16 // CK

    @functools.partial(
        pl.kernel,
        out_type=jax.ShapeDtypeStruct((4, ACCQ, 2 * HD), jnp.float32),
        mesh=_sc_mesh(),
        scratch_types=[
            pltpu.VMEM((CK,), jnp.int32),
            pltpu.VMEM((CK,), jnp.int32),
            pltpu.VMEM((CK,), jnp.int32),
            pltpu.VMEM((CK, 2 * HD), jnp.float32),
            pltpu.VMEM((ZS, 2 * HD), jnp.float32),
            pltpu.VMEM_SHARED((ACCQ, 2 * HD), jnp.float32),
            pltpu.SemaphoreType.DMA,
        ],
    )
    def k(table, rowp, colp, zw, out,
          srcv, dstv, locv, rows, stage, acc, sem):
        c = lax.axis_index("c")
        s = lax.axis_index("s")
        for r in range(2):
            base_n = c * (2 * NCQ) + r * NCQ
            _q_zero(zw, stage, acc, s)
            plsc.subcore_barrier()

            def body(i, carry):
                e0 = s * ep16 + i * CK
                pltpu.sync_copy(rowp.at[pl.ds(e0, CK)], srcv)
                pltpu.sync_copy(colp.at[pl.ds(e0, CK)], dstv)
                pltpu.async_copy(table.at[srcv], rows, sem).wait()
                _loc_idx(dstv, locv, base_n, NCQ, TRQ)
                pltpu.sync_copy(rows, acc.at[locv], add=True)
                return carry

            lax.fori_loop(0, nch, body, 0)
            plsc.subcore_barrier()
            _q_out(acc, stage, out, 2 * c + r, s)
            plsc.subcore_barrier()

    return k


def _make_att(ep):
    ep16 = ep // 16
    nch = ep16 // CK

    @functools.partial(
        pl.kernel,
        out_type=jax.ShapeDtypeStruct((ep * 16,), jnp.float32),
        mesh=_sc_mesh(),
        scratch_types=[
            pltpu.VMEM((CK,), jnp.int32),
            pltpu.VMEM((CK,), jnp.int32),
            pltpu.VMEM((CK, 128), jnp.float32),
            pltpu.VMEM((CK, 128), jnp.float32),
            pltpu.VMEM((CK * 16,), jnp.float32),
            pltpu.SemaphoreType.DMA,
            pltpu.SemaphoreType.DMA,
        ],
    )
    def k(asf, adf, rowp, colp, exo,
          srcv, dstv, sr, dr, exv, sem1, sem2):
        c = lax.axis_index("c")
        s = lax.axis_index("s")
        half = nch // 2

        def body(i, carry):
            e0 = s * ep16 + i * CK
            pltpu.sync_copy(rowp.at[pl.ds(e0, CK)], srcv)
            pltpu.sync_copy(colp.at[pl.ds(e0, CK)], dstv)
            cp1 = pltpu.async_copy(asf.at[srcv], sr, sem1)
            cp2 = pltpu.async_copy(adf.at[dstv], dr, sem2)
            cp1.wait()
            cp2.wait()
            for e in range(CK):
                a = sr[e, pl.ds(0, 16)] + dr[e, pl.ds(0, 16)]
                a = jnp.maximum(a, 0.2 * a)
                exv[pl.ds(e * 16, 16)] = jnp.exp(a)
            pltpu.sync_copy(exv, exo.at[pl.ds(e0 * 16, CK * 16)])
            return carry

        lax.fori_loop(c * half, c * half + half, body, 0)

    return k


def _make_gat(ep, h, n):
    ep16 = ep // 16
    nch = ep16 // CK

    @functools.partial(
        pl.kernel,
        out_type=jax.ShapeDtypeStruct((4, ACCQ, 2 * HD), jnp.float32),
        mesh=_sc_mesh(),
        scratch_types=[
            pltpu.VMEM((CK,), jnp.int32),
            pltpu.VMEM((CK,), jnp.int32),
            pltpu.VMEM((CK,), jnp.int32),
            pltpu.VMEM((CK,), jnp.int32),
            pltpu.VMEM((CK, 2 * HD), jnp.float32),
            pltpu.VMEM((CK * 16,), jnp.float32),
            pltpu.VMEM((ZS, 2 * HD), jnp.float32),
            pltpu.VMEM_SHARED((ACCQ, 2 * HD), jnp.float32),
            pltpu.SemaphoreType.DMA,
        ],
    )
    def k(hwf, rowp, colp, ex, zw, out,
          srcv, dstv, locv, gidx, rows, exb, stage, acc, sem):
        c = lax.axis_index("c")
        s = lax.axis_index("s")
        for r in range(2):
            base_n = c * (2 * NCQ) + r * NCQ
            _q_zero(zw, stage, acc, s)
            plsc.subcore_barrier()

            def body(i, carry):
                e0 = s * ep16 + i * CK
                pltpu.sync_copy(rowp.at[pl.ds(e0, CK)], srcv)
                pltpu.sync_copy(colp.at[pl.ds(e0, CK)], dstv)
                pltpu.sync_copy(ex.at[pl.ds(e0 * 16, CK * 16)], exb)
                for j in range(_G16):
                    gidx[pl.ds(j * 16, 16)] = srcv[pl.ds(j * 16, 16)] + h * n
                pltpu.async_copy(hwf.at[gidx], rows, sem).wait()
                _loc_idx(dstv, locv, base_n, NCQ, TRQ)
                for e in range(CK):
                    ev = exb[pl.ds(e * 16, 16)]
                    sp = jnp.full((16,), ev[h])
                    for q in range(HD // 16):
                        rows[e, pl.ds(q * 16, 16)] = (
                            rows[e, pl.ds(q * 16, 16)] * sp)
                    rows[e, pl.ds(HD, 16)] = ev
                pltpu.sync_copy(rows, acc.at[locv], add=True)
                return carry

            lax.fori_loop(0, nch, body, 0)
            plsc.subcore_barrier()
            _q_out(acc, stage, out, 2 * c + r, s)
            plsc.subcore_barrier()

    return k


def _wide(t):
    return jnp.concatenate([t, jnp.zeros_like(t)], axis=1)


def _t1_body(x_ref, w_ref, deg_ref, t_ref, dinv_ref):
    dinv = lax.rsqrt(deg_ref[...])
    t_ref[...] = _wide(dinv * jnp.dot(x_ref[...], w_ref[...],
                                      preferred_element_type=jnp.float32))
    dinv_ref[...] = dinv


def _t2_body(agg_ref, dinv_ref, b_ref, w_ref, h_ref, t_ref):
    dinv = dinv_ref[...]
    h = jnp.maximum(dinv * agg_ref[...] + b_ref[...], 0.0)
    h_ref[...] = h
    t_ref[...] = _wide(dinv * jnp.dot(h, w_ref[...],
                                      preferred_element_type=jnp.float32))


def _t3_body(hp_ref, agg_ref, dinv_ref, b_ref, w_ref, h_ref, t_ref):
    dinv = dinv_ref[...]
    h = hp_ref[...] + jnp.maximum(dinv * agg_ref[...] + b_ref[...], 0.0)
    h_ref[...] = h
    t_ref[...] = _wide(dinv * jnp.dot(h, w_ref[...],
                                      preferred_element_type=jnp.float32))


def _t4_body(hp_ref, agg_ref, dinv_ref, b_ref, wg_ref, as_ref, ad_ref,
             hw_ref, sa_ref, da_ref):
    dinv = dinv_ref[...]
    h = hp_ref[...] + jnp.maximum(dinv * agg_ref[...] + b_ref[...], 0.0)
    hw = jnp.dot(h, wg_ref[0], preferred_element_type=jnp.float32)
    hw_ref[...] = _wide(hw)
    sa_ref[...] = jnp.sum(hw * as_ref[0], axis=1, keepdims=True)[None]
    da_ref[...] = jnp.sum(hw * ad_ref[0], axis=1, keepdims=True)[None]


def _t5_body(g0, g1, g2, g3, den_ref, bg_ref, wc1_ref, bc1_ref,
             wc2_ref, bc2_ref, out_ref):
    den = den_ref[...] + 1e-16
    h_att = jnp.concatenate(
        [g[...] / den[:, hh:hh + 1] for hh, g in enumerate((g0, g1, g2, g3))],
        axis=1) + bg_ref[...]
    z = jnp.maximum(jnp.dot(h_att, wc1_ref[...],
                            preferred_element_type=jnp.float32) + bc1_ref[...], 0.0)
    logits = jnp.dot(z, wc2_ref[...],
                     preferred_element_type=jnp.float32) + bc2_ref[...]
    m = jnp.max(logits, axis=1, keepdims=True)
    lse = m + jnp.log(jnp.sum(jnp.exp(logits - m), axis=1, keepdims=True))
    out_ref[...] = logits - lse


def _rows(shape2):
    return pl.BlockSpec((BN, shape2), lambda i: (i, 0))


def _full(s0, s1):
    return pl.BlockSpec((s0, s1), lambda i: (0, 0))


def _half_cat(o):
    return jnp.concatenate([o[0, :NCK], o[1, :NCK]], axis=0)


def _quad_cat(o, n):
    return jnp.concatenate([o[q, :NCQ] for q in range(4)], axis=0)[:n]


def kernel(x, edge_index, W1, b1, W2, b2, W3, b3, Wg, att_src, att_dst, bg,
           Wc1, bc1, Wc2, bc2):
    n = x.shape[0]
    e = edge_index.shape[1]
    grid = (n // BN,)

    ar = jnp.arange(n, dtype=jnp.int32)
    ep = ((e + n + 16 * CK - 1) // (16 * CK)) * (16 * CK)
    pad = ep - (e + n)
    rowp = jnp.concatenate([edge_index[0], ar, jnp.zeros((pad,), jnp.int32)])
    colp = jnp.concatenate([edge_index[1], ar, jnp.full((pad,), 2 * n, jnp.int32)])
    z16 = jnp.zeros((ZS, 16), jnp.float32)
    zw = jnp.zeros((ZS, 2 * HD), jnp.float32)

    deg4 = _make_deg(ep)(colp, z16)
    deg = _quad_cat(deg4, n)[:, :1]

    t1, dinv = pl.pallas_call(
        _t1_body, grid=grid,
        in_specs=[_rows(8), _full(8, HD), _rows(1)],
        out_specs=[_rows(2 * HD), _rows(1)],
        out_shape=[jax.ShapeDtypeStruct((n, 2 * HD), jnp.float32),
                   jax.ShapeDtypeStruct((n, 1), jnp.float32)],
    )(x, W1, deg)

    seg = _make_seg(ep)
    agg1 = _quad_cat(seg(t1, rowp, colp, zw), n)[:, :HD]
    h1, t2 = pl.pallas_call(
        _t2_body, grid=grid,
        in_specs=[_rows(HD), _rows(1), _full(1, HD), _full(HD, HD)],
        out_specs=[_rows(HD), _rows(2 * HD)],
        out_shape=[jax.ShapeDtypeStruct((n, HD), jnp.float32),
                   jax.ShapeDtypeStruct((n, 2 * HD), jnp.float32)],
    )(agg1, dinv, b1.reshape(1, HD), W2)

    agg2 = _quad_cat(seg(t2, rowp, colp, zw), n)[:, :HD]
    h2, t3 = pl.pallas_call(
        _t3_body, grid=grid,
        in_specs=[_rows(HD), _rows(HD), _rows(1), _full(1, HD), _full(HD, HD)],
        out_specs=[_rows(HD), _rows(2 * HD)],
        out_shape=[jax.ShapeDtypeStruct((n, HD), jnp.float32),
                   jax.ShapeDtypeStruct((n, 2 * HD), jnp.float32)],
    )(h1, agg2, dinv, b2.reshape(1, HD), W3)

    agg3 = _quad_cat(seg(t3, rowp, colp, zw), n)[:, :HD]

    gh = (n // BN, NH)
    hwf, a_s3, a_d3 = pl.pallas_call(
        _t4_body, grid=gh,
        in_specs=[
            pl.BlockSpec((BN, HD), lambda i, h: (i, 0)),
            pl.BlockSpec((BN, HD), lambda i, h: (i, 0)),
            pl.BlockSpec((BN, 1), lambda i, h: (i, 0)),
            pl.BlockSpec((1, HD), lambda i, h: (0, 0)),
            pl.BlockSpec((1, HD, HD), lambda i, h: (h, 0, 0)),
            pl.BlockSpec((1, 1, HD), lambda i, h: (h, 0, 0)),
            pl.BlockSpec((1, 1, HD), lambda i, h: (h, 0, 0)),
        ],
        out_specs=[
            pl.BlockSpec((BN, 2 * HD), lambda i, h: (h * (n // BN) + i, 0)),
            pl.BlockSpec((1, BN, 1), lambda i, h: (h, i, 0)),
            pl.BlockSpec((1, BN, 1), lambda i, h: (h, i, 0)),
        ],
        out_shape=[jax.ShapeDtypeStruct((NH * n, 2 * HD), jnp.float32),
                   jax.ShapeDtypeStruct((NH, n, 1), jnp.float32),
                   jax.ShapeDtypeStruct((NH, n, 1), jnp.float32)],
    )(h2, agg3, dinv, b3.reshape(1, HD),
      Wg.reshape(HD, NH, HD).transpose(1, 0, 2),
      att_src.reshape(1, NH, HD).transpose(1, 0, 2),
      att_dst.reshape(1, NH, HD).transpose(1, 0, 2))

    asw = jnp.pad(a_s3[:, :, 0].transpose(1, 0), ((0, 0), (0, 128 - NH)))
    adw = jnp.pad(a_d3[:, :, 0].transpose(1, 0), ((0, 0), (0, 128 - NH)))
    ex = _make_att(ep)(asw, adw, rowp, colp)

    gouts = [_quad_cat(_make_gat(ep, h, n)(hwf, rowp, colp, ex, zw), n)
             for h in range(NH)]
    gats = [g[:, :HD] for g in gouts]
    den = gouts[0][:, HD:HD + NH]

    out = pl.pallas_call(
        _t5_body, grid=grid,
        in_specs=[_rows(HD), _rows(HD), _rows(HD), _rows(HD), _rows(NH),
                  _full(1, NH * HD), _full(NH * HD, HD), _full(1, HD),
                  _full(HD, 3), _full(1, 3)],
        out_specs=_rows(3),
        out_shape=jax.ShapeDtypeStruct((n, 3), jnp.float32),
    )(gats[0], gats[1], gats[2], gats[3], den, bg.reshape(1, NH * HD),
      Wc1, bc1.reshape(1, HD), Wc2, bc2.reshape(1, 3))
    return out
